# baseline (speedup 1.0000x reference)
import functools

import jax
import jax.numpy as jnp
import ml_dtypes
import numpy as np
from jax import lax
from jax.experimental import pallas as pl
from jax.experimental.pallas import tpu as pltpu
from jax.experimental.pallas import tpu_sc as plsc

EPS = 1e-06
NUM_CLASSES = 10
MAX_IMVOTE = 3
IMG_H, IMG_W = 480, 600
B = 2
SEED_NUM = 16384
BBOX_NUM = 128
HW = IMG_H * IMG_W


def _bf(v):
    return float(np.asarray(v, np.float32).astype(ml_dtypes.bfloat16).astype(np.float32))


_C530 = _bf(530.0)
_C300 = _bf(300.0)
_C240 = _bf(240.0)
_CINV = _bf(np.float32(1.0) / np.float32(530.0))

S_BLK = 2048

_SC_CORES = 2
_SC_SUBCORES = 16
_NW = _SC_CORES * _SC_SUBCORES
_NTOT = B * SEED_NUM
_PER_W = _NTOT // _NW
_WPB = _NW // B
_ROWS_PER_CH = HW // 8
_CHUNKS = _PER_W // 16


def _project(seeds_ref):
    x = seeds_ref[0, 0:1, :]
    y = seeds_ref[0, 1:2, :]
    z = seeds_ref[0, 2:3, :]
    xb = x.astype(jnp.bfloat16).astype(jnp.float32)
    yb = y.astype(jnp.bfloat16).astype(jnp.float32)
    zb = z.astype(jnp.bfloat16).astype(jnp.float32)
    p0 = _C530 * xb + _C300 * yb
    p1 = _C240 * yb - _C530 * zb
    uo = jnp.round(p0 / yb - 1.0)
    vo = jnp.round(p1 / yb - 1.0)
    return x, y, z, uo, vo, yb


def _pix_body(seeds_ref, pix_ref):
    _, _, _, uo, vo, _ = _project(seeds_ref)
    u_cl = jnp.clip(uo, 0.0, IMG_W - 1.0)
    v_cl = jnp.clip(vo, 0.0, IMG_H - 1.0)
    pix_ref[0, 0:1, :] = (v_cl * IMG_W + u_cl).astype(jnp.int32)


def _pix_call(seeds_t):
    return pl.pallas_call(
        _pix_body,
        grid=(B, SEED_NUM // S_BLK),
        in_specs=[pl.BlockSpec((1, 3, S_BLK), lambda b, i: (b, 0, i))],
        out_specs=pl.BlockSpec((1, 1, S_BLK), lambda b, i: (b, 0, i)),
        out_shape=jax.ShapeDtypeStruct((B, 1, SEED_NUM), jnp.int32),
        compiler_params=pltpu.CompilerParams(
            dimension_semantics=("parallel", "parallel")),
    )(seeds_t)


_sc_mesh = plsc.VectorSubcoreMesh(core_axis_name="c", subcore_axis_name="s")


@functools.partial(
    pl.kernel,
    mesh=_sc_mesh,
    compiler_params=pltpu.CompilerParams(use_tc_tiling_on_sc=False,
                                         needs_layout_passes=False),
    out_type=jax.ShapeDtypeStruct((B, 8, SEED_NUM), jnp.float32),
    scratch_types=[
        pltpu.VMEM((_PER_W,), jnp.int32),
        pltpu.VMEM((_PER_W,), jnp.int32),
        pltpu.VMEM((_PER_W,), jnp.int32),
        pltpu.VMEM((_PER_W, 8), jnp.float32),
        pltpu.VMEM((_PER_W,), jnp.float32),
        pltpu.SemaphoreType.DMA,
    ],
)
def _sc_gather(img_hbm, pix_hbm, out_hbm, pix_v, row_v, lane_v, rows_v, txt_v, sem):
    wid = lax.axis_index("s") * _SC_CORES + lax.axis_index("c")
    b = wid // _WPB
    s0 = (wid % _WPB) * _PER_W
    pltpu.sync_copy(pix_hbm.at[pl.ds(wid * _PER_W, _PER_W)], pix_v)

    def split_body(j, _):
        o = pl.multiple_of(j * 16, 16)
        p = pix_v[pl.ds(o, 16)]
        row_v[pl.ds(o, 16)] = lax.shift_right_logical(p, 3)
        lane_v[pl.ds(o, 16)] = lax.bitwise_and(p, 7)
        return 0

    lax.fori_loop(0, _CHUNKS, split_body, 0)

    for c in range(3):
        ch_base = (b * 3 + c) * _ROWS_PER_CH

        def off_body(j, _):
            o = pl.multiple_of(j * 16, 16)
            pix_v[pl.ds(o, 16)] = row_v[pl.ds(o, 16)] + ch_base
            return 0

        lax.fori_loop(0, _CHUNKS, off_body, 0)
        pltpu.async_copy(img_hbm.at[pix_v], rows_v, sem).wait()

        def sel_body(j, _):
            o = pl.multiple_of(j * 16, 16)
            ridx = lax.iota(jnp.int32, 16) + j * 16
            vals = plsc.load_gather(rows_v, [ridx, lane_v[pl.ds(o, 16)]])
            txt_v[pl.ds(o, 16)] = vals / 255.0
            return 0

        lax.fori_loop(0, _CHUNKS, sel_body, 0)
        pltpu.sync_copy(txt_v, out_hbm.at[b, c, pl.ds(s0, _PER_W)])


def _tc_body(seeds_ref, bbox_ref, txt_ref, feat_ref, mask_ref,
             cue_sc, msk_sc):
    k_id = pl.program_id(2)

    @pl.when(k_id > 0)
    def _replay():
        feat_ref[0, 0:18, :] = cue_sc[k_id - 1]
        mask_ref[0, 0:1, :] = msk_sc[k_id - 1]

    @pl.when(k_id == 0)
    def _compute():
        _tc_compute(seeds_ref, bbox_ref, txt_ref, feat_ref, mask_ref,
                    cue_sc, msk_sc)


def _tc_compute(seeds_ref, bbox_ref, txt_ref, feat_ref, mask_ref,
                cue_sc, msk_sc):
    x, y, z, uo, vo, zc = _project(seeds_ref)

    bl = bbox_ref[0, :, 0:1]
    bt = bbox_ref[0, :, 1:2]
    br = bbox_ref[0, :, 2:3]
    bb = bbox_ref[0, :, 3:4]
    bconf = bbox_ref[0, :, 4:5]

    in_bbox = (uo > bl) & (uo < br) & (vo > bt) & (vo < bb)
    score = in_bbox.astype(jnp.float32) + bconf

    iota_b = lax.broadcasted_iota(jnp.int32, (BBOX_NUM, S_BLK), 0)
    iota_c = lax.broadcasted_iota(jnp.int32, (NUM_CLASSES, S_BLK), 0)
    params_t = bbox_ref[0, :, :]

    for k in range(MAX_IMVOTE):
        m = jnp.max(score, axis=0, keepdims=True)
        sel_i = jnp.min(jnp.where(score == m, iota_b, BBOX_NUM),
                        axis=0, keepdims=True)
        onehot = (iota_b == sel_i).astype(jnp.float32)
        selp = lax.dot_general(params_t, onehot, (((0,), (0,)), ((), ())),
                               precision=lax.Precision.HIGHEST)
        sbl = selp[0:1, :]
        sbt = selp[1:2, :]
        sbr = selp[2:3, :]
        sbb = selp[3:4, :]
        sconf = selp[4:5, :]
        scls = selp[5:6, :]

        inb = (uo > sbl) & (uo < sbr) & (vo > sbt) & (vo < sbb)
        inbf = inb.astype(jnp.float32)

        du = (sbl + sbr) * 0.5 - uo
        dv = (sbt + sbb) * 0.5 - vo
        v0 = (du * zc).astype(jnp.bfloat16).astype(jnp.float32)
        v1 = (dv * zc).astype(jnp.bfloat16).astype(jnp.float32)
        iv0 = v0 * _CINV
        iv2 = -(v1 * _CINV)
        r0 = x + iv0
        r1 = y
        r2 = z + iv2
        rn = jnp.sqrt(r0 * r0 + r1 * r1 + r2 * r2 + EPS)
        r0 = r0 / rn
        r1 = r1 / rn
        r2 = r2 / rn
        xz0 = r0 / (r1 + EPS) * y - x
        xz1 = r2 / (r1 + EPS) * y - z

        cls_i = scls.astype(jnp.int32)
        sem = (iota_c == cls_i).astype(jnp.float32) * sconf

        geo = jnp.concatenate([xz0, xz1, r0, r1, r2], axis=0)
        cue = jnp.concatenate([geo, sem], axis=0) * inbf
        txt3 = txt_ref[0, 0:3, :]
        if k == 0:
            feat_ref[0, 0:15, :] = cue
            feat_ref[0, 15:18, :] = txt3
            mask_ref[0, 0:1, :] = inb.astype(jnp.int32)
        else:
            cue_sc[k - 1, 0:15, :] = cue
            cue_sc[k - 1, 15:18, :] = txt3
            msk_sc[k - 1, 0:1, :] = inb.astype(jnp.int32)

        score = jnp.where(iota_b == sel_i, -1.0, score)


_NBLK = SEED_NUM // S_BLK


def _tc_call(seeds_t, bboxes, txt):
    return pl.pallas_call(
        _tc_body,
        grid=(B, _NBLK, MAX_IMVOTE),
        in_specs=[
            pl.BlockSpec((1, 3, S_BLK), lambda b, i, k: (b, 0, i)),
            pl.BlockSpec((1, BBOX_NUM, 6), lambda b, i, k: (b, 0, 0)),
            pl.BlockSpec((1, 8, S_BLK), lambda b, i, k: (b, 0, i)),
        ],
        out_specs=[
            pl.BlockSpec((1, 18, S_BLK), lambda b, i, k: (b, 0, k * _NBLK + i)),
            pl.BlockSpec((1, 1, S_BLK), lambda b, i, k: (b, 0, k * _NBLK + i)),
        ],
        out_shape=[
            jax.ShapeDtypeStruct((B, 18, MAX_IMVOTE * SEED_NUM), jnp.float32),
            jax.ShapeDtypeStruct((B, 1, MAX_IMVOTE * SEED_NUM), jnp.int32),
        ],
        scratch_shapes=[
            pltpu.VMEM((MAX_IMVOTE - 1, 18, S_BLK), jnp.float32),
            pltpu.VMEM((MAX_IMVOTE - 1, 1, S_BLK), jnp.int32),
        ],
        compiler_params=pltpu.CompilerParams(
            dimension_semantics=("parallel", "parallel", "arbitrary")),
    )(seeds_t, bboxes, txt)


def kernel(imgs, bboxes_2d_rescaled, seeds_3d_depth):
    seeds_t = jnp.transpose(seeds_3d_depth, (0, 2, 1))
    pix = _pix_call(seeds_t)
    img_rows = imgs.reshape(B * 3 * _ROWS_PER_CH, 8)
    txt = _sc_gather(img_rows, pix.reshape(_NTOT))
    feats, mask_i = _tc_call(seeds_t, bboxes_2d_rescaled, txt)
    return feats, mask_i.astype(bool).reshape(B, MAX_IMVOTE * SEED_NUM)


# speedup vs baseline: 1.1965x; 1.1965x over previous
import functools

import jax
import jax.numpy as jnp
import ml_dtypes
import numpy as np
from jax import lax
from jax.experimental import pallas as pl
from jax.experimental.pallas import tpu as pltpu
from jax.experimental.pallas import tpu_sc as plsc

EPS = 1e-06
NUM_CLASSES = 10
MAX_IMVOTE = 3
IMG_H, IMG_W = 480, 600
B = 2
SEED_NUM = 16384
BBOX_NUM = 128
HW = IMG_H * IMG_W


def _bf(v):
    return float(np.asarray(v, np.float32).astype(ml_dtypes.bfloat16).astype(np.float32))


_C530 = _bf(530.0)
_C300 = _bf(300.0)
_C240 = _bf(240.0)
_CINV = _bf(np.float32(1.0) / np.float32(530.0))

S_BLK = 2048

_SC_CORES = 2
_SC_SUBCORES = 16
_NW = _SC_CORES * _SC_SUBCORES
_NTOT = B * SEED_NUM
_PER_W = _NTOT // _NW
_WPB = _NW // B
_ROWS_PER_CH = HW // 8
_CHUNKS = _PER_W // 16


def _project(seeds_ref):
    x = seeds_ref[0, 0:1, :]
    y = seeds_ref[0, 1:2, :]
    z = seeds_ref[0, 2:3, :]
    xb = x.astype(jnp.bfloat16).astype(jnp.float32)
    yb = y.astype(jnp.bfloat16).astype(jnp.float32)
    zb = z.astype(jnp.bfloat16).astype(jnp.float32)
    p0 = _C530 * xb + _C300 * yb
    p1 = _C240 * yb - _C530 * zb
    uo = jnp.round(p0 / yb - 1.0)
    vo = jnp.round(p1 / yb - 1.0)
    return x, y, z, uo, vo, yb


def _pix_body(seeds_ref, pix_ref):
    _, _, _, uo, vo, _ = _project(seeds_ref)
    u_cl = jnp.clip(uo, 0.0, IMG_W - 1.0)
    v_cl = jnp.clip(vo, 0.0, IMG_H - 1.0)
    pix_ref[0, 0:1, :] = (v_cl * IMG_W + u_cl).astype(jnp.int32)


def _pix_call(seeds_t):
    return pl.pallas_call(
        _pix_body,
        grid=(B, SEED_NUM // S_BLK),
        in_specs=[pl.BlockSpec((1, 3, S_BLK), lambda b, i: (b, 0, i))],
        out_specs=pl.BlockSpec((1, 1, S_BLK), lambda b, i: (b, 0, i)),
        out_shape=jax.ShapeDtypeStruct((B, 1, SEED_NUM), jnp.int32),
        compiler_params=pltpu.CompilerParams(
            dimension_semantics=("parallel", "parallel")),
    )(seeds_t)


_sc_mesh = plsc.VectorSubcoreMesh(core_axis_name="c", subcore_axis_name="s")


@functools.partial(
    pl.kernel,
    mesh=_sc_mesh,
    compiler_params=pltpu.CompilerParams(use_tc_tiling_on_sc=False,
                                         needs_layout_passes=False),
    out_type=jax.ShapeDtypeStruct((B, 8, SEED_NUM), jnp.float32),
    scratch_types=[
        pltpu.VMEM((_PER_W,), jnp.int32),
        pltpu.VMEM((_PER_W,), jnp.int32),
        pltpu.VMEM((_PER_W,), jnp.int32),
        pltpu.VMEM((_PER_W, 8), jnp.float32),
        pltpu.VMEM((_PER_W,), jnp.float32),
        pltpu.SemaphoreType.DMA,
    ],
)
def _sc_gather(img_hbm, pix_hbm, out_hbm, pix_v, row_v, lane_v, rows_v, txt_v, sem):
    wid = lax.axis_index("s") * _SC_CORES + lax.axis_index("c")
    b = wid // _WPB
    s0 = (wid % _WPB) * _PER_W
    pltpu.sync_copy(pix_hbm.at[pl.ds(wid * _PER_W, _PER_W)], pix_v)

    def split_body(j, _):
        o = pl.multiple_of(j * 16, 16)
        p = pix_v[pl.ds(o, 16)]
        row_v[pl.ds(o, 16)] = lax.shift_right_logical(p, 3)
        lane_v[pl.ds(o, 16)] = lax.bitwise_and(p, 7)
        return 0

    lax.fori_loop(0, _CHUNKS, split_body, 0)

    for c in range(3):
        ch_base = (b * 3 + c) * _ROWS_PER_CH

        def off_body(j, _):
            o = pl.multiple_of(j * 16, 16)
            pix_v[pl.ds(o, 16)] = row_v[pl.ds(o, 16)] + ch_base
            return 0

        lax.fori_loop(0, _CHUNKS, off_body, 0)
        pltpu.async_copy(img_hbm.at[pix_v], rows_v, sem).wait()

        def sel_body(j, _):
            o = pl.multiple_of(j * 16, 16)
            ridx = lax.iota(jnp.int32, 16) + j * 16
            vals = plsc.load_gather(rows_v, [ridx, lane_v[pl.ds(o, 16)]])
            txt_v[pl.ds(o, 16)] = vals / 255.0
            return 0

        lax.fori_loop(0, _CHUNKS, sel_body, 0)
        pltpu.sync_copy(txt_v, out_hbm.at[b, c, pl.ds(s0, _PER_W)])


def _tc_body(seeds_ref, bbox_ref, txt_ref, feat_ref, mask_ref):
    i_id = pl.program_id(1)
    col0 = pl.multiple_of(i_id * S_BLK, S_BLK)
    x, y, z, uo, vo, zc = _project(seeds_ref)

    bl = bbox_ref[0, :, 0:1]
    bt = bbox_ref[0, :, 1:2]
    br = bbox_ref[0, :, 2:3]
    bb = bbox_ref[0, :, 3:4]
    bconf = bbox_ref[0, :, 4:5]

    in_bbox = (uo > bl) & (uo < br) & (vo > bt) & (vo < bb)
    score = in_bbox.astype(jnp.float32) + bconf

    iota_b = lax.broadcasted_iota(jnp.int32, (BBOX_NUM, S_BLK), 0)
    iota_c = lax.broadcasted_iota(jnp.int32, (NUM_CLASSES, S_BLK), 0)
    params_t = bbox_ref[0, :, :]

    for k in range(MAX_IMVOTE):
        m = jnp.max(score, axis=0, keepdims=True)
        sel_i = jnp.min(jnp.where(score == m, iota_b, BBOX_NUM),
                        axis=0, keepdims=True)
        onehot = (iota_b == sel_i).astype(jnp.float32)
        selp = lax.dot_general(params_t, onehot, (((0,), (0,)), ((), ())),
                               precision=lax.Precision.HIGHEST)
        sbl = selp[0:1, :]
        sbt = selp[1:2, :]
        sbr = selp[2:3, :]
        sbb = selp[3:4, :]
        sconf = selp[4:5, :]
        scls = selp[5:6, :]

        inb = (uo > sbl) & (uo < sbr) & (vo > sbt) & (vo < sbb)
        inbf = inb.astype(jnp.float32)

        du = (sbl + sbr) * 0.5 - uo
        dv = (sbt + sbb) * 0.5 - vo
        v0 = (du * zc).astype(jnp.bfloat16).astype(jnp.float32)
        v1 = (dv * zc).astype(jnp.bfloat16).astype(jnp.float32)
        iv0 = v0 * _CINV
        iv2 = -(v1 * _CINV)
        r0 = x + iv0
        r1 = y
        r2 = z + iv2
        rn = jnp.sqrt(r0 * r0 + r1 * r1 + r2 * r2 + EPS)
        r0 = r0 / rn
        r1 = r1 / rn
        r2 = r2 / rn
        xz0 = r0 / (r1 + EPS) * y - x
        xz1 = r2 / (r1 + EPS) * y - z

        cls_i = scls.astype(jnp.int32)
        sem = (iota_c == cls_i).astype(jnp.float32) * sconf

        geo = jnp.concatenate([xz0, xz1, r0, r1, r2], axis=0)
        cue = jnp.concatenate([geo, sem], axis=0) * inbf
        txt3 = txt_ref[0, 0:3, :]
        feat_ref[0, 0:15, pl.ds(col0 + k * SEED_NUM, S_BLK)] = cue
        feat_ref[0, 15:18, pl.ds(col0 + k * SEED_NUM, S_BLK)] = txt3
        mask_ref[0, 0:1, pl.ds(col0 + k * SEED_NUM, S_BLK)] = inb.astype(jnp.int32)

        score = jnp.where(iota_b == sel_i, -1.0, score)


_NBLK = SEED_NUM // S_BLK


def _tc_call(seeds_t, bboxes, txt):
    return pl.pallas_call(
        _tc_body,
        grid=(B, _NBLK),
        in_specs=[
            pl.BlockSpec((1, 3, S_BLK), lambda b, i: (b, 0, i)),
            pl.BlockSpec((1, BBOX_NUM, 6), lambda b, i: (b, 0, 0)),
            pl.BlockSpec((1, 8, S_BLK), lambda b, i: (b, 0, i)),
        ],
        out_specs=[
            pl.BlockSpec((1, 18, MAX_IMVOTE * SEED_NUM), lambda b, i: (b, 0, 0)),
            pl.BlockSpec((1, 1, MAX_IMVOTE * SEED_NUM), lambda b, i: (b, 0, 0)),
        ],
        out_shape=[
            jax.ShapeDtypeStruct((B, 18, MAX_IMVOTE * SEED_NUM), jnp.float32),
            jax.ShapeDtypeStruct((B, 1, MAX_IMVOTE * SEED_NUM), jnp.int32),
        ],
        compiler_params=pltpu.CompilerParams(
            dimension_semantics=("parallel", "arbitrary")),
    )(seeds_t, bboxes, txt)


def kernel(imgs, bboxes_2d_rescaled, seeds_3d_depth):
    seeds_t = jnp.transpose(seeds_3d_depth, (0, 2, 1))
    pix = _pix_call(seeds_t)
    img_rows = imgs.reshape(B * 3 * _ROWS_PER_CH, 8)
    txt = _sc_gather(img_rows, pix.reshape(_NTOT))
    feats, mask_i = _tc_call(seeds_t, bboxes_2d_rescaled, txt)
    return feats, mask_i.astype(bool).reshape(B, MAX_IMVOTE * SEED_NUM)


# speedup vs baseline: 1.1987x; 1.0018x over previous
import functools

import jax
import jax.numpy as jnp
import ml_dtypes
import numpy as np
from jax import lax
from jax.experimental import pallas as pl
from jax.experimental.pallas import tpu as pltpu
from jax.experimental.pallas import tpu_sc as plsc

EPS = 1e-06
NUM_CLASSES = 10
MAX_IMVOTE = 3
IMG_H, IMG_W = 480, 600
B = 2
SEED_NUM = 16384
BBOX_NUM = 128
HW = IMG_H * IMG_W


def _bf(v):
    return float(np.asarray(v, np.float32).astype(ml_dtypes.bfloat16).astype(np.float32))


_C530 = _bf(530.0)
_C300 = _bf(300.0)
_C240 = _bf(240.0)
_CINV = _bf(np.float32(1.0) / np.float32(530.0))

S_BLK = 2048

_SC_CORES = 2
_SC_SUBCORES = 16
_NW = _SC_CORES * _SC_SUBCORES
_NTOT = B * SEED_NUM
_PER_W = _NTOT // _NW
_WPB = _NW // B
_ROWS_PER_CH = HW // 8
_CHUNKS = _PER_W // 16


def _project(seeds_ref):
    x = seeds_ref[0, 0:1, :]
    y = seeds_ref[0, 1:2, :]
    z = seeds_ref[0, 2:3, :]
    xb = x.astype(jnp.bfloat16).astype(jnp.float32)
    yb = y.astype(jnp.bfloat16).astype(jnp.float32)
    zb = z.astype(jnp.bfloat16).astype(jnp.float32)
    p0 = _C530 * xb + _C300 * yb
    p1 = _C240 * yb - _C530 * zb
    uo = jnp.round(p0 / yb - 1.0)
    vo = jnp.round(p1 / yb - 1.0)
    return x, y, z, uo, vo, yb


def _pix_body(seeds_ref, pix_ref):
    _, _, _, uo, vo, _ = _project(seeds_ref)
    u_cl = jnp.clip(uo, 0.0, IMG_W - 1.0)
    v_cl = jnp.clip(vo, 0.0, IMG_H - 1.0)
    pix_ref[0, 0:1, :] = (v_cl * IMG_W + u_cl).astype(jnp.int32)


def _pix_call(seeds_t):
    return pl.pallas_call(
        _pix_body,
        grid=(B, SEED_NUM // S_BLK),
        in_specs=[pl.BlockSpec((1, 3, S_BLK), lambda b, i: (b, 0, i))],
        out_specs=pl.BlockSpec((1, 1, S_BLK), lambda b, i: (b, 0, i)),
        out_shape=jax.ShapeDtypeStruct((B, 1, SEED_NUM), jnp.int32),
        compiler_params=pltpu.CompilerParams(
            dimension_semantics=("parallel", "parallel")),
    )(seeds_t)


_sc_mesh = plsc.VectorSubcoreMesh(core_axis_name="c", subcore_axis_name="s")


@functools.partial(
    pl.kernel,
    mesh=_sc_mesh,
    compiler_params=pltpu.CompilerParams(use_tc_tiling_on_sc=False,
                                         needs_layout_passes=False),
    out_type=jax.ShapeDtypeStruct((B, 8, SEED_NUM), jnp.float32),
    scratch_types=[
        pltpu.VMEM((_PER_W,), jnp.int32),
        pltpu.VMEM((_PER_W,), jnp.int32),
        pltpu.VMEM((_PER_W,), jnp.int32),
        pltpu.VMEM((_PER_W, 8), jnp.float32),
        pltpu.VMEM((_PER_W,), jnp.float32),
        pltpu.SemaphoreType.DMA,
    ],
)
def _sc_gather(img_hbm, pix_hbm, out_hbm, pix_v, row_v, lane_v, rows_v, txt_v, sem):
    wid = lax.axis_index("s") * _SC_CORES + lax.axis_index("c")
    b = wid // _WPB
    s0 = (wid % _WPB) * _PER_W
    pltpu.sync_copy(pix_hbm.at[pl.ds(wid * _PER_W, _PER_W)], pix_v)

    def split_body(j, _):
        o = pl.multiple_of(j * 16, 16)
        p = pix_v[pl.ds(o, 16)]
        row_v[pl.ds(o, 16)] = lax.shift_right_logical(p, 3)
        lane_v[pl.ds(o, 16)] = lax.bitwise_and(p, 7)
        return 0

    lax.fori_loop(0, _CHUNKS, split_body, 0)

    for c in range(3):
        ch_base = (b * 3 + c) * _ROWS_PER_CH

        def off_body(j, _):
            o = pl.multiple_of(j * 16, 16)
            pix_v[pl.ds(o, 16)] = row_v[pl.ds(o, 16)] + ch_base
            return 0

        lax.fori_loop(0, _CHUNKS, off_body, 0)
        pltpu.async_copy(img_hbm.at[pix_v], rows_v, sem).wait()

        def sel_body(j, _):
            o = pl.multiple_of(j * 16, 16)
            ridx = lax.iota(jnp.int32, 16) + j * 16
            vals = plsc.load_gather(rows_v, [ridx, lane_v[pl.ds(o, 16)]])
            txt_v[pl.ds(o, 16)] = vals / 255.0
            return 0

        lax.fori_loop(0, _CHUNKS, sel_body, 0)
        pltpu.sync_copy(txt_v, out_hbm.at[b, c, pl.ds(s0, _PER_W)])


def _tc_body(seeds_ref, bbox_ref, txt_ref, feat_ref, mask_ref):
    i_id = pl.program_id(1)
    col0 = pl.multiple_of(i_id * S_BLK, S_BLK)
    x, y, z, uo, vo, zc = _project(seeds_ref)

    bl = bbox_ref[0, :, 0:1]
    bt = bbox_ref[0, :, 1:2]
    br = bbox_ref[0, :, 2:3]
    bb = bbox_ref[0, :, 3:4]
    bconf = bbox_ref[0, :, 4:5]

    in_bbox = (uo > bl) & (uo < br) & (vo > bt) & (vo < bb)
    score = in_bbox.astype(jnp.float32) + bconf

    iota_b = lax.broadcasted_iota(jnp.int32, (BBOX_NUM, S_BLK), 0)
    iota_c = lax.broadcasted_iota(jnp.int32, (NUM_CLASSES, S_BLK), 0)
    params_t = bbox_ref[0, :, :]

    for k in range(MAX_IMVOTE):
        m = jnp.max(score, axis=0, keepdims=True)
        sel_i = jnp.min(jnp.where(score == m, iota_b, BBOX_NUM),
                        axis=0, keepdims=True)
        onehot = (iota_b == sel_i).astype(jnp.float32)
        selp = lax.dot_general(params_t, onehot, (((0,), (0,)), ((), ())),
                               precision=lax.Precision.HIGHEST)
        sbl = selp[0:1, :]
        sbt = selp[1:2, :]
        sbr = selp[2:3, :]
        sbb = selp[3:4, :]
        sconf = selp[4:5, :]
        scls = selp[5:6, :]

        inb = (uo > sbl) & (uo < sbr) & (vo > sbt) & (vo < sbb)
        inbf = inb.astype(jnp.float32)

        du = (sbl + sbr) * 0.5 - uo
        dv = (sbt + sbb) * 0.5 - vo
        v0 = (du * zc).astype(jnp.bfloat16).astype(jnp.float32)
        v1 = (dv * zc).astype(jnp.bfloat16).astype(jnp.float32)
        iv0 = v0 * _CINV
        iv2 = -(v1 * _CINV)
        r0 = x + iv0
        r1 = y
        r2 = z + iv2
        rn = jnp.sqrt(r0 * r0 + r1 * r1 + r2 * r2 + EPS)
        r0 = r0 / rn
        r1 = r1 / rn
        r2 = r2 / rn
        xz0 = r0 / (r1 + EPS) * y - x
        xz1 = r2 / (r1 + EPS) * y - z

        cls_i = scls.astype(jnp.int32)
        sem = (iota_c == cls_i).astype(jnp.float32) * sconf

        geo = jnp.concatenate([xz0, xz1, r0, r1, r2], axis=0)
        cue = jnp.concatenate([geo, sem], axis=0) * inbf
        txt3 = txt_ref[0, 0:3, :]
        feat_ref[0, 0:15, pl.ds(col0 + k * SEED_NUM, S_BLK)] = cue
        feat_ref[0, 15:18, pl.ds(col0 + k * SEED_NUM, S_BLK)] = txt3
        mask_ref[0, 0:1, pl.ds(col0 + k * SEED_NUM, S_BLK)] = inb

        score = jnp.where(iota_b == sel_i, -1.0, score)


_NBLK = SEED_NUM // S_BLK


def _tc_call(seeds_t, bboxes, txt):
    return pl.pallas_call(
        _tc_body,
        grid=(B, _NBLK),
        in_specs=[
            pl.BlockSpec((1, 3, S_BLK), lambda b, i: (b, 0, i)),
            pl.BlockSpec((1, BBOX_NUM, 6), lambda b, i: (b, 0, 0)),
            pl.BlockSpec((1, 8, S_BLK), lambda b, i: (b, 0, i)),
        ],
        out_specs=[
            pl.BlockSpec((1, 18, MAX_IMVOTE * SEED_NUM), lambda b, i: (b, 0, 0)),
            pl.BlockSpec((1, 1, MAX_IMVOTE * SEED_NUM), lambda b, i: (b, 0, 0)),
        ],
        out_shape=[
            jax.ShapeDtypeStruct((B, 18, MAX_IMVOTE * SEED_NUM), jnp.float32),
            jax.ShapeDtypeStruct((B, 1, MAX_IMVOTE * SEED_NUM), jnp.bool_),
        ],
        compiler_params=pltpu.CompilerParams(
            dimension_semantics=("parallel", "arbitrary")),
    )(seeds_t, bboxes, txt)


def kernel(imgs, bboxes_2d_rescaled, seeds_3d_depth):
    seeds_t = jnp.transpose(seeds_3d_depth, (0, 2, 1))
    pix = _pix_call(seeds_t)
    img_rows = imgs.reshape(B * 3 * _ROWS_PER_CH, 8)
    txt = _sc_gather(img_rows, pix.reshape(_NTOT))
    feats, mask_b = _tc_call(seeds_t, bboxes_2d_rescaled, txt)
    return feats, mask_b.reshape(B, MAX_IMVOTE * SEED_NUM)


# speedup vs baseline: 1.2020x; 1.0028x over previous
import functools

import jax
import jax.numpy as jnp
import ml_dtypes
import numpy as np
from jax import lax
from jax.experimental import pallas as pl
from jax.experimental.pallas import tpu as pltpu
from jax.experimental.pallas import tpu_sc as plsc

EPS = 1e-06
NUM_CLASSES = 10
MAX_IMVOTE = 3
IMG_H, IMG_W = 480, 600
B = 2
SEED_NUM = 16384
BBOX_NUM = 128
HW = IMG_H * IMG_W


def _bf(v):
    return float(np.asarray(v, np.float32).astype(ml_dtypes.bfloat16).astype(np.float32))


_C530 = _bf(530.0)
_C300 = _bf(300.0)
_C240 = _bf(240.0)
_CINV = _bf(np.float32(1.0) / np.float32(530.0))

S_BLK = 2048

_SC_CORES = 2
_SC_SUBCORES = 16
_NW = _SC_CORES * _SC_SUBCORES
_NTOT = B * SEED_NUM
_PER_W = _NTOT // _NW
_WPB = _NW // B
_ROWS_PER_CH = HW // 8
_CHUNKS = _PER_W // 16


def _project(seeds_ref):
    x = seeds_ref[0, 0:1, :]
    y = seeds_ref[0, 1:2, :]
    z = seeds_ref[0, 2:3, :]
    xb = x.astype(jnp.bfloat16).astype(jnp.float32)
    yb = y.astype(jnp.bfloat16).astype(jnp.float32)
    zb = z.astype(jnp.bfloat16).astype(jnp.float32)
    p0 = _C530 * xb + _C300 * yb
    p1 = _C240 * yb - _C530 * zb
    uo = jnp.round(p0 / yb - 1.0)
    vo = jnp.round(p1 / yb - 1.0)
    return x, y, z, uo, vo, yb


def _pix_body(seeds_ref, pix_ref):
    _, _, _, uo, vo, _ = _project(seeds_ref)
    u_cl = jnp.clip(uo, 0.0, IMG_W - 1.0)
    v_cl = jnp.clip(vo, 0.0, IMG_H - 1.0)
    pix_ref[0, 0:1, :] = (v_cl * IMG_W + u_cl).astype(jnp.int32)


def _pix_call(seeds_t):
    return pl.pallas_call(
        _pix_body,
        grid=(B, SEED_NUM // S_BLK),
        in_specs=[pl.BlockSpec((1, 3, S_BLK), lambda b, i: (b, 0, i))],
        out_specs=pl.BlockSpec((1, 1, S_BLK), lambda b, i: (b, 0, i)),
        out_shape=jax.ShapeDtypeStruct((B, 1, SEED_NUM), jnp.int32),
        compiler_params=pltpu.CompilerParams(
            dimension_semantics=("parallel", "parallel")),
    )(seeds_t)


_sc_mesh = plsc.VectorSubcoreMesh(core_axis_name="c", subcore_axis_name="s")


@functools.partial(
    pl.kernel,
    mesh=_sc_mesh,
    compiler_params=pltpu.CompilerParams(use_tc_tiling_on_sc=False,
                                         needs_layout_passes=False),
    out_type=jax.ShapeDtypeStruct((B, 8, SEED_NUM), jnp.float32),
    scratch_types=[
        pltpu.VMEM((_PER_W,), jnp.int32),
        pltpu.VMEM((_PER_W,), jnp.int32),
        pltpu.VMEM((_PER_W,), jnp.int32),
        pltpu.VMEM((_PER_W, 8), jnp.float32),
        pltpu.VMEM((_PER_W,), jnp.float32),
        pltpu.SemaphoreType.DMA,
    ],
)
def _sc_gather(img_hbm, pix_hbm, out_hbm, pix_v, row_v, lane_v, rows_v, txt_v, sem):
    wid = lax.axis_index("s") * _SC_CORES + lax.axis_index("c")
    b = wid // _WPB
    s0 = (wid % _WPB) * _PER_W
    pltpu.sync_copy(pix_hbm.at[pl.ds(wid * _PER_W, _PER_W)], pix_v)

    def split_body(j, _):
        o = pl.multiple_of(j * 16, 16)
        p = pix_v[pl.ds(o, 16)]
        row_v[pl.ds(o, 16)] = lax.shift_right_logical(p, 3)
        lane_v[pl.ds(o, 16)] = lax.bitwise_and(p, 7)
        return 0

    lax.fori_loop(0, _CHUNKS, split_body, 0)

    for c in range(3):
        ch_base = (b * 3 + c) * _ROWS_PER_CH

        def off_body(j, _):
            o = pl.multiple_of(j * 16, 16)
            pix_v[pl.ds(o, 16)] = row_v[pl.ds(o, 16)] + ch_base
            return 0

        lax.fori_loop(0, _CHUNKS, off_body, 0)
        pltpu.async_copy(img_hbm.at[pix_v], rows_v, sem).wait()

        def sel_body(j, _):
            o = pl.multiple_of(j * 16, 16)
            ridx = lax.iota(jnp.int32, 16) + j * 16
            vals = plsc.load_gather(rows_v, [ridx, lane_v[pl.ds(o, 16)]])
            txt_v[pl.ds(o, 16)] = vals / 255.0
            return 0

        lax.fori_loop(0, _CHUNKS, sel_body, 0)
        pltpu.sync_copy(txt_v, out_hbm.at[b, c, pl.ds(s0, _PER_W)])


def _tc_body(seeds_ref, bbox_ref, txt_ref, feat_ref, mask_ref):
    b_id = pl.program_id(0)
    i_id = pl.program_id(1)
    col0 = pl.multiple_of(i_id * S_BLK, S_BLK)
    x, y, z, uo, vo, zc = _project(seeds_ref)

    bl = bbox_ref[0, :, 0:1]
    bt = bbox_ref[0, :, 1:2]
    br = bbox_ref[0, :, 2:3]
    bb = bbox_ref[0, :, 3:4]
    bconf = bbox_ref[0, :, 4:5]

    in_bbox = (uo > bl) & (uo < br) & (vo > bt) & (vo < bb)
    score = in_bbox.astype(jnp.float32) + bconf

    iota_b = lax.broadcasted_iota(jnp.int32, (BBOX_NUM, S_BLK), 0)
    iota_c = lax.broadcasted_iota(jnp.int32, (NUM_CLASSES, S_BLK), 0)
    params_t = bbox_ref[0, :, :]

    for k in range(MAX_IMVOTE):
        m = jnp.max(score, axis=0, keepdims=True)
        sel_i = jnp.min(jnp.where(score == m, iota_b, BBOX_NUM),
                        axis=0, keepdims=True)
        onehot = (iota_b == sel_i).astype(jnp.float32)
        selp = lax.dot_general(params_t, onehot, (((0,), (0,)), ((), ())),
                               precision=lax.Precision.HIGHEST)
        sbl = selp[0:1, :]
        sbt = selp[1:2, :]
        sbr = selp[2:3, :]
        sbb = selp[3:4, :]
        sconf = selp[4:5, :]
        scls = selp[5:6, :]

        inb = (uo > sbl) & (uo < sbr) & (vo > sbt) & (vo < sbb)
        inbf = inb.astype(jnp.float32)

        du = (sbl + sbr) * 0.5 - uo
        dv = (sbt + sbb) * 0.5 - vo
        v0 = (du * zc).astype(jnp.bfloat16).astype(jnp.float32)
        v1 = (dv * zc).astype(jnp.bfloat16).astype(jnp.float32)
        iv0 = v0 * _CINV
        iv2 = -(v1 * _CINV)
        r0 = x + iv0
        r1 = y
        r2 = z + iv2
        rn = jnp.sqrt(r0 * r0 + r1 * r1 + r2 * r2 + EPS)
        r0 = r0 / rn
        r1 = r1 / rn
        r2 = r2 / rn
        xz0 = r0 / (r1 + EPS) * y - x
        xz1 = r2 / (r1 + EPS) * y - z

        cls_i = scls.astype(jnp.int32)
        sem = (iota_c == cls_i).astype(jnp.float32) * sconf

        geo = jnp.concatenate([xz0, xz1, r0, r1, r2], axis=0)
        cue = jnp.concatenate([geo, sem], axis=0) * inbf
        txt3 = txt_ref[0, 0:3, :]
        feat_ref[0, 0:15, pl.ds(col0 + k * SEED_NUM, S_BLK)] = cue
        feat_ref[0, 15:18, pl.ds(col0 + k * SEED_NUM, S_BLK)] = txt3
        mask_ref[pl.ds(b_id, 1), pl.ds(col0 + k * SEED_NUM, S_BLK)] = inb

        score = jnp.where(iota_b == sel_i, -1.0, score)


_NBLK = SEED_NUM // S_BLK


def _tc_call(seeds_t, bboxes, txt):
    return pl.pallas_call(
        _tc_body,
        grid=(B, _NBLK),
        in_specs=[
            pl.BlockSpec((1, 3, S_BLK), lambda b, i: (b, 0, i)),
            pl.BlockSpec((1, BBOX_NUM, 6), lambda b, i: (b, 0, 0)),
            pl.BlockSpec((1, 8, S_BLK), lambda b, i: (b, 0, i)),
        ],
        out_specs=[
            pl.BlockSpec((1, 18, MAX_IMVOTE * SEED_NUM), lambda b, i: (b, 0, 0)),
            pl.BlockSpec((B, MAX_IMVOTE * SEED_NUM), lambda b, i: (0, 0)),
        ],
        out_shape=[
            jax.ShapeDtypeStruct((B, 18, MAX_IMVOTE * SEED_NUM), jnp.float32),
            jax.ShapeDtypeStruct((B, MAX_IMVOTE * SEED_NUM), jnp.bool_),
        ],
        compiler_params=pltpu.CompilerParams(
            dimension_semantics=("parallel", "arbitrary")),
    )(seeds_t, bboxes, txt)


def kernel(imgs, bboxes_2d_rescaled, seeds_3d_depth):
    seeds_t = jnp.transpose(seeds_3d_depth, (0, 2, 1))
    pix = _pix_call(seeds_t)
    img_rows = imgs.reshape(B * 3 * _ROWS_PER_CH, 8)
    txt = _sc_gather(img_rows, pix.reshape(_NTOT))
    feats, mask_b = _tc_call(seeds_t, bboxes_2d_rescaled, txt)
    return feats, mask_b
